# baseline (speedup 1.0000x reference)
import functools

import jax
import jax.numpy as jnp
from jax import lax
from jax.experimental import pallas as pl
from jax.experimental.pallas import tpu as pltpu
from jax.experimental.pallas import tpu_sc as plsc

B, N, D = 8, 576, 64
N_TOK = B * N
K = 8192
DP = 2 * D

TT = 512
NTOK_A = 2048
NTOK_B = N_TOK - NTOK_A
NT_A = NTOK_A // TT
NT_B = NTOK_B // TT

NC, NS, LANES = 2, 16, 16
NW = NC * NS


def _argmin_math(z, e, en_row, idx_ref):
    zn = jnp.sum(z * z, axis=1, keepdims=True)
    dot2 = lax.dot_general(z + z, e, (((1,), (1,)), ((), ())),
                           preferred_element_type=jnp.float32)
    d = (zn + en_row) - dot2
    rmin = jnp.min(d, axis=1, keepdims=True)
    ONE = jnp.int32(0x3F800000)
    col = lax.broadcasted_iota(jnp.int32, d.shape, 1)
    pseudo = lax.bitcast_convert_type(ONE + col, jnp.float32)
    cand = jnp.where(d == rmin, pseudo, jnp.float32(4.0))
    m = jnp.min(cand, axis=1)
    idx_ref[...] = lax.bitcast_convert_type(m, jnp.int32) - ONE


def _argmin_a_body(z_ref, emb_ref, idx_ref, embp_ref, en_out_ref, en_ref):
    @pl.when(pl.program_id(0) == 0)
    def _():
        e0 = emb_ref[...]
        en_ref[...] = jnp.sum(e0 * e0, axis=1)[None, :]

    _argmin_math(z_ref[...], emb_ref[...], en_ref[...], idx_ref)

    @pl.when(pl.program_id(0) == NT_A - 1)
    def _():
        embp_ref[:, :D] = emb_ref[...]
        embp_ref[:, D:] = jnp.zeros((K, D), jnp.float32)
        en_out_ref[...] = en_ref[...]


def _argmin_b_body(z_ref, emb_ref, en_in_ref, idx_ref):
    _argmin_math(z_ref[...], emb_ref[...], en_in_ref[...], idx_ref)


_tc_argmin_a = pl.pallas_call(
    _argmin_a_body,
    grid=(NT_A,),
    in_specs=[
        pl.BlockSpec((TT, D), lambda i: (i, 0)),
        pl.BlockSpec((K, D), lambda i: (0, 0)),
    ],
    out_specs=[
        pl.BlockSpec((TT,), lambda i: (i,)),
        pl.BlockSpec((K, DP), lambda i: (0, 0)),
        pl.BlockSpec((1, K), lambda i: (0, 0)),
    ],
    out_shape=[
        jax.ShapeDtypeStruct((NTOK_A,), jnp.int32),
        jax.ShapeDtypeStruct((K, DP), jnp.float32),
        jax.ShapeDtypeStruct((1, K), jnp.float32),
    ],
    scratch_shapes=[pltpu.VMEM((1, K), jnp.float32)],
)

_tc_argmin_b = pl.pallas_call(
    _argmin_b_body,
    grid=(NT_B,),
    in_specs=[
        pl.BlockSpec((TT, D), lambda i: (i + NT_A, 0)),
        pl.BlockSpec((K, D), lambda i: (0, 0)),
        pl.BlockSpec((1, K), lambda i: (0, 0)),
    ],
    out_specs=pl.BlockSpec((TT,), lambda i: (i,)),
    out_shape=jax.ShapeDtypeStruct((NTOK_B,), jnp.int32),
)


_sc_mesh = plsc.VectorSubcoreMesh(core_axis_name="c", subcore_axis_name="s")


def _make_sc_gather_loss(ntok, tok0):
    bpw = ntok // NW
    nch = 1 if bpw <= 128 else 2
    ch = bpw // nch

    scratch = []
    for _ in range(nch):
        scratch.append(pltpu.VMEM((ch,), jnp.int32))
    for _ in range(nch):
        scratch.append(pltpu.VMEM((ch, DP), jnp.float32))
    scratch += [
        pltpu.VMEM((bpw, D), jnp.float32),
        pltpu.VMEM((bpw, D), jnp.float32),
        pltpu.VMEM((LANES,), jnp.float32),
        pltpu.SemaphoreType.DMA,
    ]

    @functools.partial(
        pl.kernel,
        mesh=_sc_mesh,
        out_type=(
            jax.ShapeDtypeStruct((ntok, D), jnp.float32),
            jax.ShapeDtypeStruct((NW * LANES,), jnp.float32),
        ),
        scratch_types=scratch,
    )
    def sc_gather_loss(emb_hbm, idx_hbm, z_hbm, out_hbm, psum_hbm, *refs):
        idx_bufs = refs[:nch]
        row_bufs = refs[nch:2 * nch]
        z_v, out_v, acc_v, sem = refs[2 * nch:]

        wid = lax.axis_index("s") * NC + lax.axis_index("c")
        base = wid * bpw
        copies = []
        for c in range(nch):
            pltpu.sync_copy(idx_hbm.at[pl.ds(base + c * ch, ch)], idx_bufs[c])
            copies.append(pltpu.async_copy(emb_hbm.at[idx_bufs[c]],
                                           row_bufs[c], sem))
        pltpu.sync_copy(z_hbm.at[pl.ds(tok0 + base, bpw)], z_v)
        for cp in copies:
            cp.wait()

        def make_body(rows, roff):
            def body(r, acc):
                for c in range(D // LANES):
                    q = rows[r - roff, pl.ds(c * LANES, LANES)]
                    t = z_v[r, pl.ds(c * LANES, LANES)]
                    out_v[r, pl.ds(c * LANES, LANES)] = q
                    dd = q - t
                    acc = acc + dd * dd
                return acc
            return body

        acc = jnp.zeros((LANES,), jnp.float32)
        for c in range(nch):
            acc = lax.fori_loop(c * ch, (c + 1) * ch,
                                make_body(row_bufs[c], c * ch), acc)
        acc_v[...] = acc

        pltpu.sync_copy(out_v, out_hbm.at[pl.ds(base, bpw)])
        pltpu.sync_copy(acc_v, psum_hbm.at[pl.ds(wid * LANES, LANES)])

    return sc_gather_loss


_sc_gather_a = _make_sc_gather_loss(NTOK_A, 0)
_sc_gather_b = _make_sc_gather_loss(NTOK_B, NTOK_A)


def kernel(z, emb_weight):
    z = z.astype(jnp.float32)
    zf = z.reshape(-1, D)
    idx_a, emb_p, en_row = _tc_argmin_a(zf, emb_weight)
    quant_a, psums_a = _sc_gather_a(emb_p, idx_a, zf)
    idx_b = _tc_argmin_b(zf, emb_weight, en_row)
    quant_b, psums_b = _sc_gather_b(emb_p, idx_b, zf)
    quantized = jnp.concatenate([quant_a, quant_b], axis=0).reshape(z.shape)
    mse = (jnp.sum(psums_a) + jnp.sum(psums_b)) / float(N_TOK * D)
    zero = jnp.array(0.0, dtype=jnp.float32)
    loss = 0.25 * mse + 1.0 * mse + 0.0 * zero
    q_indices = jnp.concatenate([idx_a, idx_b]).reshape(B, N)
    return (z, emb_weight, quantized, q_indices, loss, mse, mse,
            zero, zero, zero)


# speedup vs baseline: 1.1958x; 1.1958x over previous
import functools

import jax
import jax.numpy as jnp
from jax import lax
from jax.experimental import pallas as pl
from jax.experimental.pallas import tpu as pltpu
from jax.experimental.pallas import tpu_sc as plsc

B, N, D = 8, 576, 64
N_TOK = B * N
K = 8192
DP = 2 * D

TT = 512
NT = N_TOK // TT
CB = 512
NCH = K // CB
CB = 512
NCH = K // CB

NC, NS, LANES = 2, 16, 16
NW = NC * NS
BPW = N_TOK // NW
CH = BPW // 2


ONE = 0x3F800000


def _argmin_body(z_ref, emb_ref, idx_ref, embp_ref, en_ref, d_ref, ps_ref):
    @pl.when(pl.program_id(0) == 0)
    def _():
        e0 = emb_ref[...]
        en_ref[...] = jnp.sum(e0 * e0, axis=1)[None, :]
        col = lax.broadcasted_iota(jnp.int32, (1, K), 1)
        ps_ref[...] = lax.bitcast_convert_type(jnp.int32(ONE) + col,
                                               jnp.float32)

    z = z_ref[...]
    zn = jnp.sum(z * z, axis=1, keepdims=True)
    z2x = z + z
    mins = []
    for k in range(NCH):
        ek = emb_ref[pl.ds(k * CB, CB), :]
        enk = en_ref[:, pl.ds(k * CB, CB)]
        dot2 = lax.dot_general(z2x, ek, (((1,), (1,)), ((), ())),
                               preferred_element_type=jnp.float32)
        dk = (zn + enk) - dot2
        d_ref[:, pl.ds(k * CB, CB)] = dk
        mins.append(jnp.min(dk, axis=1, keepdims=True))
    rmin = functools.reduce(jnp.minimum, mins)
    cand = jnp.where(d_ref[...] == rmin, ps_ref[...], jnp.float32(4.0))
    m = jnp.min(cand, axis=1)
    idx_ref[...] = lax.bitcast_convert_type(m, jnp.int32) - jnp.int32(ONE)

    @pl.when(pl.program_id(0) == NT - 1)
    def _():
        embp_ref[:, :D] = emb_ref[...]
        embp_ref[:, D:] = jnp.zeros((K, D), jnp.float32)


_tc_argmin = pl.pallas_call(
    _argmin_body,
    grid=(NT,),
    in_specs=[
        pl.BlockSpec((TT, D), lambda i: (i, 0)),
        pl.BlockSpec((K, D), lambda i: (0, 0)),
    ],
    out_specs=[
        pl.BlockSpec((TT,), lambda i: (i,)),
        pl.BlockSpec((K, DP), lambda i: (0, 0)),
    ],
    out_shape=[
        jax.ShapeDtypeStruct((N_TOK,), jnp.int32),
        jax.ShapeDtypeStruct((K, DP), jnp.float32),
    ],
    scratch_shapes=[
        pltpu.VMEM((1, K), jnp.float32),
        pltpu.VMEM((TT, K), jnp.float32),
        pltpu.VMEM((1, K), jnp.float32),
    ],
)


_sc_mesh = plsc.VectorSubcoreMesh(core_axis_name="c", subcore_axis_name="s")


@functools.partial(
    pl.kernel,
    mesh=_sc_mesh,
    out_type=(
        jax.ShapeDtypeStruct((N_TOK, D), jnp.float32),
        jax.ShapeDtypeStruct((NW * LANES,), jnp.float32),
    ),
    scratch_types=[
        pltpu.VMEM((CH,), jnp.int32),
        pltpu.VMEM((CH,), jnp.int32),
        pltpu.VMEM((CH, DP), jnp.float32),
        pltpu.VMEM((CH, DP), jnp.float32),
        pltpu.VMEM((BPW, D), jnp.float32),
        pltpu.VMEM((BPW, D), jnp.float32),
        pltpu.VMEM((LANES,), jnp.float32),
        pltpu.SemaphoreType.DMA,
    ],
)
def _sc_gather_loss(emb_hbm, idx_hbm, z_hbm, out_hbm, psum_hbm,
                    idx_a, idx_b, rows_a, rows_b, z_v, out_v, acc_v, sem):
    wid = lax.axis_index("s") * NC + lax.axis_index("c")
    base = wid * BPW
    pltpu.sync_copy(idx_hbm.at[pl.ds(base, CH)], idx_a)
    pltpu.sync_copy(idx_hbm.at[pl.ds(base + CH, CH)], idx_b)
    ca = pltpu.async_copy(emb_hbm.at[idx_a], rows_a, sem)
    cb = pltpu.async_copy(emb_hbm.at[idx_b], rows_b, sem)
    pltpu.sync_copy(z_hbm.at[pl.ds(base, BPW)], z_v)
    ca.wait()
    cb.wait()

    def make_body(rows, roff):
        def body(r, acc):
            for c in range(D // LANES):
                q = rows[r - roff, pl.ds(c * LANES, LANES)]
                t = z_v[r, pl.ds(c * LANES, LANES)]
                out_v[r, pl.ds(c * LANES, LANES)] = q
                dd = q - t
                acc = acc + dd * dd
            return acc
        return body

    acc = lax.fori_loop(0, CH, make_body(rows_a, 0),
                        jnp.zeros((LANES,), jnp.float32))
    acc = lax.fori_loop(CH, BPW, make_body(rows_b, CH), acc)
    acc_v[...] = acc

    pltpu.sync_copy(out_v, out_hbm.at[pl.ds(base, BPW)])
    pltpu.sync_copy(acc_v, psum_hbm.at[pl.ds(wid * LANES, LANES)])


def kernel(z, emb_weight):
    z = z.astype(jnp.float32)
    zf = z.reshape(-1, D)
    idx_flat, emb_p = _tc_argmin(zf, emb_weight)
    quant_flat, psums = _sc_gather_loss(emb_p, idx_flat, zf)
    quantized = quant_flat.reshape(z.shape)
    mse = jnp.sum(psums) / float(N_TOK * D)
    zero = jnp.array(0.0, dtype=jnp.float32)
    loss = 0.25 * mse + 1.0 * mse + 0.0 * zero
    q_indices = idx_flat.reshape(B, N)
    return (z, emb_weight, quantized, q_indices, loss, mse, mse,
            zero, zero, zero)
